# baseline (speedup 1.0000x reference)
import functools

import jax
import jax.numpy as jnp
import numpy as np
from jax import lax
from jax.experimental import pallas as pl
from jax.experimental.pallas import tpu as pltpu
from jax.experimental.pallas import tpu_sc as plsc

NC = 2
NS = 16
L = 16

F = 9
V = 1000

NB = 32
WB = NB * F
FRAME = F * L
VPB = WB // L
GROUPS = 3
GROW = WB // GROUPS

INT_MAX = 2147483647

_FEAT = np.arange(FRAME, dtype=np.int32) % F


def _adjust_body(x_ref, adj_ref):
  m = jnp.min(x_ref[...], axis=0, keepdims=True)
  feat = lax.broadcasted_iota(jnp.int32, (1, FRAME), 1) % F
  adj = jnp.zeros((1, FRAME), jnp.int32)
  for fi in range(F):
    mask = feat == fi
    mfi = jnp.min(jnp.where(mask, m, INT_MAX), axis=1, keepdims=True)
    adj = jnp.where(mask, V * fi - mfi, adj)
  adj_ref[...] = adj


def _sc_body(NBLK, KMAX, xflat, adj_hbm, comb, out,
             xbuf0, adjbuf, idxbuf0, gbuf0,
             obuf0, obuf1, stab, gsem0, osem0, osem1):
  c = lax.axis_index("c")
  s = lax.axis_index("s")
  wid = s * NC + c

  TR = (F * V // NS) & ~7
  pltpu.sync_copy(comb.at[pl.ds(s * TR, TR)], stab.at[pl.ds(s * TR, TR)])
  REM = F * V - NS * TR

  if REM > 0:
    @pl.when(s == 0)
    def _():
      pltpu.sync_copy(comb.at[pl.ds(NS * TR, REM)],
                      stab.at[pl.ds(NS * TR, REM)])

  plsc.subcore_barrier()

  obufs = (obuf0, obuf1)
  osems = (osem0, osem1)

  pltpu.sync_copy(adj_hbm, adjbuf)

  CHUNK_HI = [(GROW * (g + 1)) // F for g in range(GROUPS)]
  CHUNK_LO = [0] + CHUNK_HI[:-1]

  def step(k, p):
    jc = wid + 32 * k

    @pl.when(jc < NBLK)
    def _():
      pltpu.sync_copy(xflat.at[pl.ds(jc * WB, WB)], xbuf0)
      for v in range(VPB):
        idx = xbuf0[pl.ds(v * L, L)] + adjbuf[0, pl.ds((v % F) * L, L)]
        idxbuf0[v // (GROW // L), pl.ds((v % (GROW // L)) * L, L)] = idx
      for g in range(GROUPS):
        pltpu.async_copy(stab.at[idxbuf0.at[g]],
                         gbuf0.at[pl.ds(g * GROW, GROW)], gsem0)

      @pl.when(k >= 2)
      def _():
        pltpu.make_async_copy(obufs[p],
                              out.at[pl.ds((jc - 64) * NB, NB)],
                              osems[p]).wait()

      obuf = obufs[p]
      for g in range(GROUPS):
        pltpu.make_async_copy(stab.at[pl.ds(0, GROW)],
                              gbuf0.at[pl.ds(g * GROW, GROW)], gsem0).wait()

        def node_body(n, _):
          for cc in range(8):
            acc = gbuf0[n * F, pl.ds(cc * L, L)]
            for fi in range(1, F):
              acc = acc + gbuf0[n * F + fi, pl.ds(cc * L, L)]
            obuf[n, pl.ds(cc * L, L)] = acc
          return 0

        lax.fori_loop(CHUNK_LO[g], CHUNK_HI[g], node_body, 0)

      pltpu.async_copy(obuf, out.at[pl.ds(jc * NB, NB)], osems[p])

  def pair_body(m, _):
    step(2 * m, 0)
    step(2 * m + 1, 1)
    return 0

  lax.fori_loop(0, KMAX // 2, pair_body, 0)

  kstar = (NBLK - 1 - wid) // 32
  for p in (0, 1):
    kp_last = kstar - ((kstar - p) % 2)

    @pl.when(kp_last >= 0)
    def _():
      pltpu.make_async_copy(obufs[p],
                            out.at[pl.ds((wid + 32 * kp_last) * NB, NB)],
                            osems[p]).wait()


@functools.partial(jax.jit, static_argnums=(2, 3))
def _run(xflat, comb, N, D):
  NBLK = N // NB
  KMAX = (NBLK + 31) // 32

  adj = pl.pallas_call(
      _adjust_body,
      out_shape=jax.ShapeDtypeStruct((1, FRAME), jnp.int32),
  )(xflat.reshape(-1, FRAME))

  mesh = plsc.VectorSubcoreMesh(core_axis_name="c", subcore_axis_name="s")
  body = functools.partial(_sc_body, NBLK, KMAX)
  xflat = pltpu.with_memory_space_constraint(xflat, pltpu.HBM)
  return pl.kernel(
      body,
      out_type=jax.ShapeDtypeStruct((N, D), jnp.float32),
      mesh=mesh,
      scratch_types=[
          pltpu.VMEM((WB,), jnp.int32),
          pltpu.VMEM((1, FRAME), jnp.int32),
          pltpu.VMEM((GROUPS, GROW), jnp.int32),
          pltpu.VMEM((WB, 128), jnp.float32),
          pltpu.VMEM((NB, 128), jnp.float32),
          pltpu.VMEM((NB, 128), jnp.float32),
          pltpu.VMEM_SHARED((F * V, 128), jnp.float32),
          pltpu.SemaphoreType.DMA,
          pltpu.SemaphoreType.DMA,
          pltpu.SemaphoreType.DMA,
      ],
  )(xflat, adj, comb)


def kernel(x, tables):
  N = x.shape[0]
  D = tables.shape[-1]
  xflat = x.reshape(-1)
  comb = tables.reshape(-1, D)
  return _run(xflat, comb, N, D)


# speedup vs baseline: 1.0343x; 1.0343x over previous
import functools

import jax
import jax.numpy as jnp
from jax import lax
from jax.experimental import pallas as pl
from jax.experimental.pallas import tpu as pltpu
from jax.experimental.pallas import tpu_sc as plsc

NC = 2
NS = 16
L = 16

F = 9
V = 1000

NB = 16
WB = NB * F
FRAME = F * L
VPB = WB // L
RA = 48
RB = WB - RA
VA = RA // L

INT_MAX = 2147483647


def _adjust_body(x_ref, adj_ref):
  m = jnp.min(x_ref[...], axis=0, keepdims=True)
  feat = lax.broadcasted_iota(jnp.int32, (1, FRAME), 1) % F
  adj = jnp.zeros((1, FRAME), jnp.int32)
  for fi in range(F):
    mask = feat == fi
    mfi = jnp.min(jnp.where(mask, m, INT_MAX), axis=1, keepdims=True)
    adj = jnp.where(mask, V * fi - mfi, adj)
  adj_ref[...] = adj


def _sc_body(NBLK, KMAX, xflat, adj_hbm, comb, out,
             xbuf0, xbuf1, adjbuf, idxa0, idxa1, idxb0, idxb1,
             gbuf0, gbuf1, obuf0, obuf1, stab,
             gsem0, gsem1, hsem0, hsem1, osem0, osem1):
  c = lax.axis_index("c")
  s = lax.axis_index("s")
  wid = s * NC + c

  xbufs = (xbuf0, xbuf1)
  idxas = (idxa0, idxa1)
  idxbs = (idxb0, idxb1)
  gbufs = (gbuf0, gbuf1)
  obufs = (obuf0, obuf1)
  gsems = (gsem0, gsem1)
  hsems = (hsem0, hsem1)
  osems = (osem0, osem1)

  TR = (F * V // NS) & ~7
  pltpu.sync_copy(comb.at[pl.ds(s * TR, TR)], stab.at[pl.ds(s * TR, TR)])
  REM = F * V - NS * TR

  if REM > 0:
    @pl.when(s == 0)
    def _():
      pltpu.sync_copy(comb.at[pl.ds(NS * TR, REM)],
                      stab.at[pl.ds(NS * TR, REM)])

  plsc.subcore_barrier()

  pltpu.sync_copy(adj_hbm, adjbuf)

  def prep(j, p):
    xbuf, idxa, idxb = xbufs[p], idxas[p], idxbs[p]
    pltpu.sync_copy(xflat.at[pl.ds(j * WB, WB)], xbuf)
    for v in range(VPB):
      idx = xbuf[pl.ds(v * L, L)] + adjbuf[0, pl.ds(v * L, L)]
      if v < VA:
        idxa[pl.ds(v * L, L)] = idx
      else:
        idxb[pl.ds((v - VA) * L, L)] = idx
    pltpu.async_copy(stab.at[idxa], gbufs[p].at[pl.ds(0, RA)], gsems[p])
    pltpu.async_copy(comb.at[idxb], gbufs[p].at[pl.ds(RA, RB)], hsems[p])

  def wait_gathers(p):
    pltpu.make_async_copy(stab.at[pl.ds(0, RA)],
                          gbufs[p].at[pl.ds(0, RA)], gsems[p]).wait()
    pltpu.make_async_copy(comb.at[pl.ds(0, RB)],
                          gbufs[p].at[pl.ds(RA, RB)], hsems[p]).wait()

  def accum_and_out(j, p):
    gbuf, obuf = gbufs[p], obufs[p]

    def node_body(n, _):
      for cc in range(8):
        acc = gbuf[n * F, pl.ds(cc * L, L)]
        for fi in range(1, F):
          acc = acc + gbuf[n * F + fi, pl.ds(cc * L, L)]
        obuf[n, pl.ds(cc * L, L)] = acc
      return 0

    lax.fori_loop(0, NB, node_body, 0)
    pltpu.async_copy(obuf, out.at[pl.ds(j * NB, NB)], osems[p])

  def step(k, p):
    jn = wid + 32 * (k + 1)

    @pl.when(jn < NBLK)
    def _():
      prep(jn, 1 - p)

    jc = wid + 32 * k

    @pl.when(jc < NBLK)
    def _():
      wait_gathers(p)

      @pl.when(k >= 2)
      def _():
        pltpu.make_async_copy(obufs[p],
                              out.at[pl.ds((jc - 64) * NB, NB)],
                              osems[p]).wait()

      accum_and_out(jc, p)

  prep(wid, 0)

  def pair_body(m, _):
    step(2 * m, 0)
    step(2 * m + 1, 1)
    return 0

  lax.fori_loop(0, KMAX // 2, pair_body, 0)

  kstar = (NBLK - 1 - wid) // 32
  for p in (0, 1):
    kp_last = kstar - ((kstar - p) % 2)

    @pl.when(kp_last >= 0)
    def _():
      pltpu.make_async_copy(obufs[p],
                            out.at[pl.ds((wid + 32 * kp_last) * NB, NB)],
                            osems[p]).wait()


@functools.partial(jax.jit, static_argnums=(2, 3))
def _run(xflat, comb, N, D):
  NBLK = N // NB
  KMAX = (NBLK + 31) // 32
  if KMAX % 2:
    KMAX += 1

  adj = pl.pallas_call(
      _adjust_body,
      out_shape=jax.ShapeDtypeStruct((1, FRAME), jnp.int32),
  )(xflat.reshape(-1, FRAME))

  mesh = plsc.VectorSubcoreMesh(core_axis_name="c", subcore_axis_name="s")
  body = functools.partial(_sc_body, NBLK, KMAX)
  return pl.kernel(
      body,
      out_type=jax.ShapeDtypeStruct((N, D), jnp.float32),
      mesh=mesh,
      scratch_types=[
          pltpu.VMEM((WB,), jnp.int32),
          pltpu.VMEM((WB,), jnp.int32),
          pltpu.VMEM((1, FRAME), jnp.int32),
          pltpu.VMEM((RA,), jnp.int32),
          pltpu.VMEM((RA,), jnp.int32),
          pltpu.VMEM((RB,), jnp.int32),
          pltpu.VMEM((RB,), jnp.int32),
          pltpu.VMEM((WB, 128), jnp.float32),
          pltpu.VMEM((WB, 128), jnp.float32),
          pltpu.VMEM((NB, 128), jnp.float32),
          pltpu.VMEM((NB, 128), jnp.float32),
          pltpu.VMEM_SHARED((F * V, 128), jnp.float32),
          pltpu.SemaphoreType.DMA,
          pltpu.SemaphoreType.DMA,
          pltpu.SemaphoreType.DMA,
          pltpu.SemaphoreType.DMA,
          pltpu.SemaphoreType.DMA,
          pltpu.SemaphoreType.DMA,
      ],
  )(xflat, adj, comb)


def kernel(x, tables):
  N = x.shape[0]
  D = tables.shape[-1]
  xflat = x.reshape(-1)
  comb = tables.reshape(-1, D)
  return _run(xflat, comb, N, D)
